# baseline (speedup 1.0000x reference)
import functools

import jax
import jax.numpy as jnp
from jax import lax
from jax.experimental import pallas as pl
from jax.experimental.pallas import tpu as pltpu
from jax.experimental.pallas import tpu_sc as plsc

D = 16
B = 16384
L = 16


def kernel(src, dst, h_output, node_biases, mu):
    info = plsc.get_sparse_core_info()
    NC, NS = info.num_cores, info.num_subcores
    NW = NC * NS
    BPW = B // NW

    mesh = plsc.VectorSubcoreMesh(core_axis_name="c", subcore_axis_name="s")

    @functools.partial(
        pl.kernel,
        out_type=jax.ShapeDtypeStruct((B,), jnp.float32),
        mesh=mesh,
        compiler_params=pltpu.CompilerParams(
            needs_layout_passes=False, use_tc_tiling_on_sc=False),
        scratch_types=[
            pltpu.VMEM((BPW,), jnp.int32),
            pltpu.VMEM((BPW,), jnp.int32),
            pltpu.VMEM((BPW,), jnp.int32),
            pltpu.VMEM((BPW,), jnp.int32),
            pltpu.VMEM((D, BPW), jnp.float32),
            pltpu.VMEM((D, BPW), jnp.float32),
            pltpu.VMEM((BPW,), jnp.float32),
            pltpu.VMEM((BPW,), jnp.float32),
            pltpu.VMEM((BPW,), jnp.float32),
            pltpu.VMEM((L,), jnp.float32),
            pltpu.SemaphoreType.DMA,
        ],
    )
    def body(src_hbm, dst_hbm, ht_hbm, nb_hbm, mu_hbm, out_hbm,
             sidx, didx, sp1, dp1, gs, gd, bs, bd, ob, muv, sem):
        wid = lax.axis_index("s") * NC + lax.axis_index("c")
        base = wid * BPW

        pltpu.sync_copy(src_hbm.at[pl.ds(base, BPW)], sidx)
        pltpu.sync_copy(dst_hbm.at[pl.ds(base, BPW)], didx)
        pltpu.sync_copy(mu_hbm, muv)

        copies = []
        for j in range(D):
            copies.append(pltpu.async_copy(
                ht_hbm.at[j].at[sidx], gs.at[j], sem))
            copies.append(pltpu.async_copy(
                ht_hbm.at[j].at[didx], gd.at[j], sem))

        def addone(i, _):
            sl = pl.ds(i * L, L)
            sp1[sl] = sidx[sl] + 1
            dp1[sl] = didx[sl] + 1
            return 0
        lax.fori_loop(0, BPW // L, addone, 0)

        copies.append(pltpu.async_copy(nb_hbm.at[sp1], bs, sem))
        copies.append(pltpu.async_copy(nb_hbm.at[dp1], bd, sem))

        for cp in copies:
            cp.wait()

        mu0 = muv[...][0]

        def chunk_body(cidx, _):
            sl = pl.ds(cidx * L, L)
            acc = bs[sl] + bd[sl] + mu0
            for j in range(D):
                acc = acc + gs[j, sl] * gd[j, sl]
            ob[sl] = acc
            return 0
        lax.fori_loop(0, BPW // L, chunk_body, 0)

        pltpu.sync_copy(ob, out_hbm.at[pl.ds(base, BPW)])

    mu16 = jnp.broadcast_to(mu, (L,))
    return body(src, dst, h_output.T, node_biases, mu16)


# speedup vs baseline: 16.8595x; 16.8595x over previous
import functools

import jax
import jax.numpy as jnp
from jax import lax
from jax.experimental import pallas as pl
from jax.experimental.pallas import tpu as pltpu
from jax.experimental.pallas import tpu_sc as plsc

N = 1000000
D = 16
B = 16384
L = 16
NP = 8


def kernel(src, dst, h_output, node_biases, mu):
    info = plsc.get_sparse_core_info()
    NC, NS = info.num_cores, info.num_subcores
    SPT = B // NS

    mesh = plsc.VectorSubcoreMesh(core_axis_name="c", subcore_axis_name="s")

    @functools.partial(
        pl.kernel,
        out_type=jax.ShapeDtypeStruct((2 * B,), jnp.float32),
        mesh=mesh,
        compiler_params=pltpu.CompilerParams(
            needs_layout_passes=False, use_tc_tiling_on_sc=True),
        scratch_types=[
            pltpu.VMEM((SPT,), jnp.int32),
            pltpu.VMEM((SPT,), jnp.int32),
            pltpu.VMEM((SPT,), jnp.int32),
            pltpu.VMEM((SPT,), jnp.int32),
            pltpu.VMEM((SPT,), jnp.float32),
            pltpu.VMEM((SPT,), jnp.float32),
            pltpu.VMEM((SPT,), jnp.float32),
            pltpu.VMEM((SPT,), jnp.float32),
            pltpu.VMEM((SPT,), jnp.float32),
            pltpu.VMEM((L,), jnp.float32),
            pltpu.VMEM_SHARED((N,), jnp.float32),
            pltpu.SemaphoreType.DMA,
            pltpu.SemaphoreType.DMA,
            pltpu.SemaphoreType.DMA,
        ],
    )
    def body(src_hbm, dst_hbm, ht_hbm, nb_hbm, mu_hbm, out_hbm,
             sidx, didx, sp1, dp1, gs, gd, acc, bs, bd, muv,
             pA, semp, semg, semb):
        c = lax.axis_index("c")
        s = lax.axis_index("s")
        sbase = s * SPT
        jbase = c * NP

        pltpu.sync_copy(src_hbm.at[pl.ds(sbase, SPT)], sidx)
        pltpu.sync_copy(dst_hbm.at[pl.ds(sbase, SPT)], didx)
        pltpu.sync_copy(mu_hbm, muv)

        def addone(i, _):
            sl = pl.ds(i * L, L)
            sp1[sl] = sidx[sl] + 1
            dp1[sl] = didx[sl] + 1
            return 0
        lax.fori_loop(0, SPT // L, addone, 0)

        @pl.when(c == 0)
        def _():
            pltpu.async_copy(nb_hbm.at[sp1], bs, semb)
            pltpu.async_copy(nb_hbm.at[dp1], bd, semb)

        for jj in range(NP):
            @pl.when(s == 0)
            def _(jj=jj):
                pltpu.async_copy(ht_hbm.at[jbase + jj], pA, semp)
                pltpu.make_async_copy(ht_hbm.at[jbase + jj], pA, semp).wait()

            plsc.subcore_barrier()

            g1 = pltpu.async_copy(pA.at[sidx], gs, semg)
            g2 = pltpu.async_copy(pA.at[didx], gd, semg)
            g1.wait()
            g2.wait()

            def accum(i, _, first=(jj == 0)):
                sl = pl.ds(i * L, L)
                prod = gs[sl] * gd[sl]
                acc[sl] = prod if first else acc[sl] + prod
                return 0
            lax.fori_loop(0, SPT // L, accum, 0)

            plsc.subcore_barrier()

        @pl.when(c == 0)
        def _():
            pltpu.make_async_copy(nb_hbm.at[sp1], bs, semb).wait()
            pltpu.make_async_copy(nb_hbm.at[dp1], bd, semb).wait()
            mu0 = muv[...][0]

            def add_bias(i, _):
                sl = pl.ds(i * L, L)
                acc[sl] = acc[sl] + bs[sl] + bd[sl] + mu0
                return 0
            lax.fori_loop(0, SPT // L, add_bias, 0)

        pltpu.sync_copy(acc, out_hbm.at[pl.ds(c * B + sbase, SPT)])

    mu16 = jnp.broadcast_to(mu, (L,))
    parts = body(src, dst, h_output.T, node_biases, mu16)
    return parts[:B] + parts[B:]


# speedup vs baseline: 19.0751x; 1.1314x over previous
import functools

import jax
import jax.numpy as jnp
from jax import lax
from jax.experimental import pallas as pl
from jax.experimental.pallas import tpu as pltpu
from jax.experimental.pallas import tpu_sc as plsc

N = 1000000
D = 16
B = 16384
L = 16
NP = 8


def kernel(src, dst, h_output, node_biases, mu):
    info = plsc.get_sparse_core_info()
    NC, NS = info.num_cores, info.num_subcores
    SPT = B // NS

    mesh = plsc.VectorSubcoreMesh(core_axis_name="c", subcore_axis_name="s")

    @functools.partial(
        pl.kernel,
        out_type=jax.ShapeDtypeStruct((2 * B,), jnp.float32),
        mesh=mesh,
        compiler_params=pltpu.CompilerParams(
            needs_layout_passes=False, use_tc_tiling_on_sc=True),
        scratch_types=[
            pltpu.VMEM((SPT,), jnp.int32),
            pltpu.VMEM((SPT,), jnp.int32),
            pltpu.VMEM((SPT,), jnp.float32),
            pltpu.VMEM((SPT,), jnp.float32),
            pltpu.VMEM((SPT,), jnp.float32),
            pltpu.VMEM((L,), jnp.float32),
            pltpu.VMEM_SHARED((N,), jnp.float32),
            pltpu.VMEM_SHARED((N,), jnp.float32),
            pltpu.SemaphoreType.DMA,
            pltpu.SemaphoreType.DMA,
            pltpu.SemaphoreType.DMA,
        ],
    )
    def body(src_hbm, dst_hbm, ht_hbm, nb_hbm, mu_hbm, out_hbm,
             sidx, didx, gs, gd, acc, muv,
             pA, pB, semp, semg, semb):
        c = lax.axis_index("c")
        s = lax.axis_index("s")
        sbase = s * SPT
        jbase = c * NP
        bufs = [pA, pB]

        pltpu.sync_copy(src_hbm.at[pl.ds(sbase, SPT)], sidx)
        pltpu.sync_copy(dst_hbm.at[pl.ds(sbase, SPT)], didx)
        pltpu.sync_copy(mu_hbm, muv)

        @pl.when(s == 0)
        def _():
            pltpu.async_copy(ht_hbm.at[jbase], pA, semp)
            pltpu.async_copy(ht_hbm.at[jbase + 1], pB, semp)

        for jj in range(NP):
            buf = bufs[jj % 2]

            @pl.when(s == 0)
            def _(jj=jj, buf=buf):
                pltpu.make_async_copy(ht_hbm.at[jbase + jj], buf, semp).wait()

            plsc.subcore_barrier()

            g1 = pltpu.async_copy(buf.at[sidx], gs, semg)
            g2 = pltpu.async_copy(buf.at[didx], gd, semg)
            g1.wait()
            g2.wait()

            plsc.subcore_barrier()
            if jj + 2 < NP:
                @pl.when(s == 0)
                def _(jj=jj, buf=buf):
                    pltpu.async_copy(ht_hbm.at[jbase + jj + 2], buf, semp)

            def accum(i, _, first=(jj == 0)):
                sl = pl.ds(i * L, L)
                prod = gs[sl] * gd[sl]
                acc[sl] = prod if first else acc[sl] + prod
                return 0
            lax.fori_loop(0, SPT // L, accum, 0)

        @pl.when(c == 0)
        def _():
            def mkp1(i, _):
                sl = pl.ds(i * L, L)
                sidx[sl] = sidx[sl] + 1
                didx[sl] = didx[sl] + 1
                return 0
            lax.fori_loop(0, SPT // L, mkp1, 0)
            pltpu.async_copy(nb_hbm.at[sidx], gs, semb)
            pltpu.async_copy(nb_hbm.at[didx], gd, semb)
            pltpu.make_async_copy(nb_hbm.at[sidx], gs, semb).wait()
            pltpu.make_async_copy(nb_hbm.at[didx], gd, semb).wait()
            mu0 = muv[...][0]

            def add_bias(i, _):
                sl = pl.ds(i * L, L)
                acc[sl] = acc[sl] + gs[sl] + gd[sl] + mu0
                return 0
            lax.fori_loop(0, SPT // L, add_bias, 0)

        pltpu.sync_copy(acc, out_hbm.at[pl.ds(c * B + sbase, SPT)])

    mu16 = jnp.broadcast_to(mu, (L,))
    parts = body(src, dst, h_output.T, node_biases, mu16)
    return parts[:B] + parts[B:]
